# baseline (speedup 1.0000x reference)
import jax
import jax.numpy as jnp
from jax import lax
from jax.experimental import pallas as pl
from jax.experimental.pallas import tpu as pltpu
from jax.experimental.pallas import tpu_sc as plsc

BATCH = 4096
HIST = 50
HIST_PAD = 56
D_MODEL = 512

NUM_CORES = 2
NUM_SUBCORES = 16
NUM_WORKERS = NUM_CORES * NUM_SUBCORES

NSPLIT = 4
CHUNK_B = BATCH // NSPLIT
B_PER_W = CHUNK_B // NUM_WORKERS

NSLOT = 3
LANES = 16


def _emb_body(idx_hbm, table_hbm, out_hbm,
              idx_v, b0, b1, b2, t0, t1, t2,
              g0, g1, g2, s0, s1, s2):
    wid = lax.axis_index("s") * NUM_CORES + lax.axis_index("c")
    base = wid * B_PER_W

    pltpu.sync_copy(idx_hbm.at[pl.ds(base * HIST_PAD, B_PER_W * HIST_PAD)], idx_v)

    bufs = (b0, b1, b2)
    tails = (t0, t1, t2)
    gsems = (g0, g1, g2)
    ssems = (s0, s1, s2)

    def _gathers(k, b):
        off = k * HIST_PAD
        return (
            pltpu.make_async_copy(
                table_hbm.at[idx_v.at[pl.ds(off, 48)]],
                bufs[b].at[pl.ds(0, 48)], gsems[b]),
            pltpu.make_async_copy(
                table_hbm.at[idx_v.at[pl.ds(off + 48, 8)]],
                tails[b], gsems[b]),
        )

    def _store(k, b):
        return pltpu.make_async_copy(bufs[b], out_hbm.at[base + k], ssems[b])

    for k in range(2):
        for op in _gathers(k, k):
            op.start()

    def _iter(k, carry):
        slot = lax.rem(k, NSLOT)

        def _run(b):
            for op in _gathers(k, b):
                op.wait()
            for r in range(2):
                for c in range(D_MODEL // LANES):
                    bufs[b][48 + r, pl.ds(c * LANES, LANES)] = (
                        tails[b][r, pl.ds(c * LANES, LANES)])
            _store(k, b).start()

            bn = (b + 2) % NSLOT

            @pl.when(k >= 1)
            def _():
                _store(k - 1, bn).wait()

            @pl.when(k + 2 < B_PER_W)
            def _():
                for op in _gathers(k + 2, bn):
                    op.start()

        for b in range(NSLOT):
            @pl.when(slot == b)
            def _(b=b):
                _run(b)

        return carry

    lax.fori_loop(0, B_PER_W, _iter, 0)

    _store(B_PER_W - 1, (B_PER_W - 1) % NSLOT).wait()


def _make_run():
    mesh = plsc.VectorSubcoreMesh(core_axis_name="c", subcore_axis_name="s")
    return pl.kernel(
        _emb_body,
        mesh=mesh,
        out_type=jax.ShapeDtypeStruct((CHUNK_B, HIST, D_MODEL), jnp.float32),
        scratch_types=[
            pltpu.VMEM((B_PER_W * HIST_PAD,), jnp.int32),
            pltpu.VMEM((HIST, D_MODEL), jnp.float32),
            pltpu.VMEM((HIST, D_MODEL), jnp.float32),
            pltpu.VMEM((HIST, D_MODEL), jnp.float32),
            pltpu.VMEM((8, D_MODEL), jnp.float32),
            pltpu.VMEM((8, D_MODEL), jnp.float32),
            pltpu.VMEM((8, D_MODEL), jnp.float32),
            pltpu.SemaphoreType.DMA,
            pltpu.SemaphoreType.DMA,
            pltpu.SemaphoreType.DMA,
            pltpu.SemaphoreType.DMA,
            pltpu.SemaphoreType.DMA,
            pltpu.SemaphoreType.DMA,
        ],
    )


@jax.jit
def _embed(idx_pad, table):
    run = _make_run()
    outs = []
    for i in range(NSPLIT):
        chunk = lax.slice_in_dim(idx_pad, i * CHUNK_B * HIST_PAD,
                                 (i + 1) * CHUNK_B * HIST_PAD)
        outs.append(run(chunk, table))
    return jnp.concatenate(outs, axis=0)


def kernel(x, table):
    idx_pad = jnp.pad(x.astype(jnp.int32), ((0, 0), (0, HIST_PAD - HIST)),
                      mode="edge")
    return _embed(idx_pad.reshape(-1), table)


# speedup vs baseline: 1.6085x; 1.6085x over previous
import jax
import jax.numpy as jnp
from jax import lax
from jax.experimental import pallas as pl
from jax.experimental.pallas import tpu as pltpu
from jax.experimental.pallas import tpu_sc as plsc

BATCH = 4096
HIST = 50
HIST_PAD = 56
D_MODEL = 512

NUM_CORES = 2
NUM_SUBCORES = 16
NUM_WORKERS = NUM_CORES * NUM_SUBCORES

NSPLIT = 1
CHUNK_B = BATCH // NSPLIT
B_PER_W = CHUNK_B // NUM_WORKERS

NSLOT = 3
LANES = 16


def _emb_body(idx_hbm, table_hbm, out_hbm,
              idx_v, b0, b1, b2, t0, t1, t2,
              g0, g1, g2, s0, s1, s2):
    wid = lax.axis_index("s") * NUM_CORES + lax.axis_index("c")
    base = wid * B_PER_W

    pltpu.sync_copy(idx_hbm.at[pl.ds(base * HIST_PAD, B_PER_W * HIST_PAD)], idx_v)

    bufs = (b0, b1, b2)
    tails = (t0, t1, t2)
    gsems = (g0, g1, g2)
    ssems = (s0, s1, s2)

    def _gathers(k, b):
        off = k * HIST_PAD
        return (
            pltpu.make_async_copy(
                table_hbm.at[idx_v.at[pl.ds(off, 48)]],
                bufs[b].at[pl.ds(0, 48)], gsems[b]),
            pltpu.make_async_copy(
                table_hbm.at[idx_v.at[pl.ds(off + 48, 8)]],
                tails[b], gsems[b]),
        )

    def _store(k, b):
        return pltpu.make_async_copy(bufs[b], out_hbm.at[base + k], ssems[b])

    for k in range(2):
        for op in _gathers(k, k):
            op.start()

    def _iter(k, carry):
        slot = lax.rem(k, NSLOT)

        def _run(b):
            for op in _gathers(k, b):
                op.wait()
            for r in range(2):
                for c in range(D_MODEL // LANES):
                    bufs[b][48 + r, pl.ds(c * LANES, LANES)] = (
                        tails[b][r, pl.ds(c * LANES, LANES)])
            _store(k, b).start()

            bn = (b + 2) % NSLOT

            @pl.when(k >= 1)
            def _():
                _store(k - 1, bn).wait()

            @pl.when(k + 2 < B_PER_W)
            def _():
                for op in _gathers(k + 2, bn):
                    op.start()

        for b in range(NSLOT):
            @pl.when(slot == b)
            def _(b=b):
                _run(b)

        return carry

    lax.fori_loop(0, B_PER_W, _iter, 0)

    _store(B_PER_W - 1, (B_PER_W - 1) % NSLOT).wait()


def _make_run():
    mesh = plsc.VectorSubcoreMesh(core_axis_name="c", subcore_axis_name="s")
    return pl.kernel(
        _emb_body,
        mesh=mesh,
        compiler_params=pltpu.CompilerParams(use_tc_tiling_on_sc=True),
        out_type=jax.ShapeDtypeStruct((CHUNK_B, HIST, D_MODEL), jnp.float32),
        scratch_types=[
            pltpu.VMEM((B_PER_W * HIST_PAD,), jnp.int32),
            pltpu.VMEM((HIST, D_MODEL), jnp.float32),
            pltpu.VMEM((HIST, D_MODEL), jnp.float32),
            pltpu.VMEM((HIST, D_MODEL), jnp.float32),
            pltpu.VMEM((8, D_MODEL), jnp.float32),
            pltpu.VMEM((8, D_MODEL), jnp.float32),
            pltpu.VMEM((8, D_MODEL), jnp.float32),
            pltpu.SemaphoreType.DMA,
            pltpu.SemaphoreType.DMA,
            pltpu.SemaphoreType.DMA,
            pltpu.SemaphoreType.DMA,
            pltpu.SemaphoreType.DMA,
            pltpu.SemaphoreType.DMA,
        ],
    )


@jax.jit
def _embed(idx_pad, table):
    run = _make_run()
    outs = []
    for i in range(NSPLIT):
        chunk = lax.slice_in_dim(idx_pad, i * CHUNK_B * HIST_PAD,
                                 (i + 1) * CHUNK_B * HIST_PAD)
        outs.append(run(chunk, table))
    return jnp.concatenate(outs, axis=0)


def kernel(x, table):
    idx_pad = jnp.pad(x.astype(jnp.int32), ((0, 0), (0, HIST_PAD - HIST)),
                      mode="edge")
    return _embed(idx_pad.reshape(-1), table)


# speedup vs baseline: 3.5572x; 2.2115x over previous
import jax
import jax.numpy as jnp
from jax import lax
from jax.experimental import pallas as pl
from jax.experimental.pallas import tpu as pltpu
from jax.experimental.pallas import tpu_sc as plsc

BATCH = 4096
HIST = 50
D_MODEL = 512
TOTAL = BATCH * HIST

NUM_CORES = 2
NUM_SUBCORES = 16
NUM_WORKERS = NUM_CORES * NUM_SUBCORES
ROWS_PER_W = TOTAL // NUM_WORKERS

CHUNK = 80
NCHUNK = ROWS_PER_W // CHUNK
NSLOT = 3


def _emb_body(idx_hbm, table_hbm, out_hbm,
              idx_v, b0, b1, b2, g0, g1, g2, s0, s1, s2):
    wid = lax.axis_index("s") * NUM_CORES + lax.axis_index("c")
    base = wid * ROWS_PER_W

    pltpu.sync_copy(idx_hbm.at[pl.ds(base, ROWS_PER_W)], idx_v)

    bufs = (b0, b1, b2)
    gsems = (g0, g1, g2)
    ssems = (s0, s1, s2)

    def _gather(k, b):
        return pltpu.make_async_copy(
            table_hbm.at[idx_v.at[pl.ds(k * CHUNK, CHUNK)]], bufs[b], gsems[b])

    def _store(k, b):
        return pltpu.make_async_copy(
            bufs[b], out_hbm.at[pl.ds(base + k * CHUNK, CHUNK)], ssems[b])

    for k in range(2):
        _gather(k, k).start()

    def _iter(k, carry):
        slot = lax.rem(k, NSLOT)

        def _run(b):
            _gather(k, b).wait()
            _store(k, b).start()

            bn = (b + 2) % NSLOT

            @pl.when(k >= 1)
            def _():
                _store(k - 1, bn).wait()

            @pl.when(k + 2 < NCHUNK)
            def _():
                _gather(k + 2, bn).start()

        for b in range(NSLOT):
            @pl.when(slot == b)
            def _(b=b):
                _run(b)

        return carry

    lax.fori_loop(0, NCHUNK, _iter, 0)

    _store(NCHUNK - 1, (NCHUNK - 1) % NSLOT).wait()


@jax.jit
def _embed(idx_flat, table):
    mesh = plsc.VectorSubcoreMesh(core_axis_name="c", subcore_axis_name="s")
    run = pl.kernel(
        _emb_body,
        mesh=mesh,
        out_type=jax.ShapeDtypeStruct((TOTAL, D_MODEL), jnp.float32),
        scratch_types=[
            pltpu.VMEM((ROWS_PER_W,), jnp.int32),
            pltpu.VMEM((CHUNK, D_MODEL), jnp.float32),
            pltpu.VMEM((CHUNK, D_MODEL), jnp.float32),
            pltpu.VMEM((CHUNK, D_MODEL), jnp.float32),
            pltpu.SemaphoreType.DMA,
            pltpu.SemaphoreType.DMA,
            pltpu.SemaphoreType.DMA,
            pltpu.SemaphoreType.DMA,
            pltpu.SemaphoreType.DMA,
            pltpu.SemaphoreType.DMA,
        ],
    )
    return run(idx_flat, table)


def kernel(x, table):
    idx_t = x.astype(jnp.int32).T.reshape(-1)
    out = _embed(idx_t, table)
    return out.reshape(HIST, BATCH, D_MODEL).transpose(1, 0, 2)
